# baseline (speedup 1.0000x reference)
import functools

import jax
import jax.numpy as jnp
from jax import lax
from jax.experimental import pallas as pl
from jax.experimental.pallas import tpu as pltpu
from jax.experimental.pallas import tpu_sc as plsc

_SIZE = 100000
_PAD = 0
_SMOOTHING = 0.1
_CONF = 1.0 - _SMOOTHING
_EPS = _SMOOTHING / (_SIZE - 2)

_B = 1024
_CB = 2048
_NCB = (_SIZE + _CB - 1) // _CB

_NC, _NS, _L = 2, 16, 16
_NW = _NC * _NS
_BPW = _B // _NW


def _sc_gather_body(t_hbm, xflat_hbm, out_hbm, t_v, idx_v, vals_v, sem):
    wid = lax.axis_index("s") * _NC + lax.axis_index("c")
    base = wid * _BPW
    pltpu.sync_copy(t_hbm.at[pl.ds(base, _BPW)], t_v)
    for c in range(_BPW // _L):
        t16 = t_v[pl.ds(c * _L, _L)]
        rows = (base + c * _L) + lax.iota(jnp.int32, _L)
        idx_v[pl.ds(c * _L, _L)] = rows * _SIZE + t16
    pltpu.async_copy(xflat_hbm.at[idx_v], vals_v, sem).wait()
    pltpu.sync_copy(vals_v, out_hbm.at[pl.ds(base, _BPW)])


_sc_gather = pl.kernel(
    _sc_gather_body,
    out_type=jax.ShapeDtypeStruct((_B,), jnp.float32),
    mesh=plsc.VectorSubcoreMesh(
        core_axis_name="c", subcore_axis_name="s", num_cores=_NC,
        num_subcores=_NS),
    scratch_types=[
        pltpu.VMEM((_BPW,), jnp.int32),
        pltpu.VMEM((_BPW,), jnp.int32),
        pltpu.VMEM((_BPW,), jnp.float32),
        pltpu.SemaphoreType.DMA,
    ],
)


def _loss_body(t_ref, g_ref, x_ref, o_ref, acc_ref):
    j = pl.program_id(0)
    x = x_ref[...]

    @pl.when(j == 0)
    def _init():
        acc_ref[...] = -x[:, 0:1]

    @pl.when(j < _NCB - 1)
    def _mid():
        acc_ref[...] += jnp.sum(x, axis=1, keepdims=True)

    @pl.when(j == _NCB - 1)
    def _last():
        col = jax.lax.broadcasted_iota(jnp.int32, (_B, _CB), 1) + j * _CB
        s = acc_ref[...] + jnp.sum(
            jnp.where(col < _SIZE, x, 0.0), axis=1, keepdims=True)
        t = t_ref[...]
        g = g_ref[...]
        c1 = _SMOOTHING * jnp.log(jnp.float32(_EPS)) + _CONF * jnp.log(
            jnp.float32(_CONF))
        row_term = c1 - _EPS * s + (_EPS - _CONF) * g
        loss = jnp.sum(jnp.where(t != _PAD, row_term, 0.0)) * (1.0 / _B)
        o_ref[0, 0] = loss


@jax.jit
def kernel(x, target):
    t32 = target.astype(jnp.int32)
    g = _sc_gather(t32, x.reshape(_B * _SIZE))
    out = pl.pallas_call(
        _loss_body,
        grid=(_NCB,),
        in_specs=[
            pl.BlockSpec((_B, 1), lambda j: (0, 0)),
            pl.BlockSpec((_B, 1), lambda j: (0, 0)),
            pl.BlockSpec((_B, _CB), lambda j: (0, j)),
        ],
        out_specs=pl.BlockSpec(memory_space=pltpu.SMEM),
        out_shape=jax.ShapeDtypeStruct((1, 1), jnp.float32),
        scratch_shapes=[pltpu.VMEM((_B, 1), jnp.float32)],
    )(t32.reshape(_B, 1), g.reshape(_B, 1), x)
    return out[0, 0]


# speedup vs baseline: 2.2112x; 2.2112x over previous
import jax
import jax.numpy as jnp
from jax.experimental import pallas as pl
from jax.experimental.pallas import tpu as pltpu

_SIZE = 100000
_PAD = 0
_SMOOTHING = 0.1
_CONF = 1.0 - _SMOOTHING
_EPS = _SMOOTHING / (_SIZE - 2)

_B = 1024
_R = 32
_NRB = _B // _R


def _loss_body(t_ref, x_ref, o_ref):
    i = pl.program_id(0)
    x = x_ref[...]
    t = t_ref[0]
    col = jax.lax.broadcasted_iota(jnp.int32, (_R, _SIZE), 1)
    s = jnp.sum(x, axis=1, keepdims=True) - x[:, 0:1]
    g = jnp.sum(jnp.where(col == t, x, 0.0), axis=1, keepdims=True)
    c1 = _SMOOTHING * jnp.log(jnp.float32(_EPS)) + _CONF * jnp.log(
        jnp.float32(_CONF))
    row_term = c1 - _EPS * s + (_EPS - _CONF) * g
    partial = jnp.sum(jnp.where(t != _PAD, row_term, 0.0)) * (1.0 / _B)

    @pl.when(i == 0)
    def _init():
        o_ref[0, 0] = 0.0

    o_ref[0, 0] += partial


@jax.jit
def kernel(x, target):
    t2 = target.astype(jnp.int32).reshape(_NRB, _R, 1)
    out = pl.pallas_call(
        _loss_body,
        grid=(_NRB,),
        in_specs=[
            pl.BlockSpec((1, _R, 1), lambda i: (i, 0, 0)),
            pl.BlockSpec((_R, _SIZE), lambda i: (i, 0)),
        ],
        out_specs=pl.BlockSpec(memory_space=pltpu.SMEM),
        out_shape=jax.ShapeDtypeStruct((1, 1), jnp.float32),
    )(t2, x)
    return out[0, 0]
